# Initial kernel scaffold; baseline (speedup 1.0000x reference)
import jax
import jax.numpy as jnp
from jax.experimental import pallas as pl


def kernel(x, table):
    raise NotImplementedError("write your pallas kernel here")


# speedup vs baseline: 3.0698x; 3.0698x over previous
import jax
import jax.numpy as jnp
from jax.experimental import pallas as pl
from jax.experimental.pallas import tpu as pltpu
from jax.experimental.pallas import tpu_sc as plsc

_EMBED_DIM = 128
_WINDOW = 128


def _sc_gather(table, flat_idx):
    num_indices = flat_idx.shape[1]
    vector_mesh = plsc.VectorSubcoreMesh(
        core_axis_name="core", subcore_axis_name="subcore"
    )

    @pl.kernel(
        out_type=jax.ShapeDtypeStruct((num_indices, _EMBED_DIM), table.dtype),
        mesh=vector_mesh,
    )
    def gather_kernel(table_hbm, idx_hbm, out_hbm):
        def body(idx_vmem, out_vmem):
            pltpu.sync_copy(table_hbm.at[idx_vmem.at[0]], out_vmem)

        pltpu.emit_pipeline(
            body,
            grid=(num_indices // _WINDOW,),
            in_specs=[
                pl.BlockSpec((1, _WINDOW), index_map=lambda i: (0, i))
            ],
            out_specs=[
                pl.BlockSpec((_WINDOW, _EMBED_DIM), index_map=lambda i: (i, 0))
            ],
            core_axis_name=("core", "subcore"),
            dimension_semantics=(pltpu.PARALLEL,),
        )(idx_hbm, out_hbm)

    return gather_kernel(table, flat_idx)


def kernel(x, table):
    idx = x[0]
    flat_idx = idx.reshape(1, -1)
    out = _sc_gather(table, flat_idx)
    return out.reshape(idx.shape[0], idx.shape[1], _EMBED_DIM)


# speedup vs baseline: 3.2546x; 1.0602x over previous
import jax
import jax.numpy as jnp
from jax.experimental import pallas as pl
from jax.experimental.pallas import tpu as pltpu
from jax.experimental.pallas import tpu_sc as plsc

_EMBED_DIM = 128
_WINDOW = 256


def _sc_gather(table, flat_idx):
    num_indices = flat_idx.shape[1]
    vector_mesh = plsc.VectorSubcoreMesh(
        core_axis_name="core", subcore_axis_name="subcore"
    )

    @pl.kernel(
        out_type=jax.ShapeDtypeStruct((num_indices, _EMBED_DIM), table.dtype),
        mesh=vector_mesh,
    )
    def gather_kernel(table_hbm, idx_hbm, out_hbm):
        def body(idx_vmem, out_vmem):
            pltpu.sync_copy(table_hbm.at[idx_vmem.at[0]], out_vmem)

        pltpu.emit_pipeline(
            body,
            grid=(num_indices // _WINDOW,),
            in_specs=[
                pl.BlockSpec((1, _WINDOW), index_map=lambda i: (0, i))
            ],
            out_specs=[
                pl.BlockSpec((_WINDOW, _EMBED_DIM), index_map=lambda i: (i, 0))
            ],
            core_axis_name=("core", "subcore"),
            dimension_semantics=(pltpu.PARALLEL,),
        )(idx_hbm, out_hbm)

    return gather_kernel(table, flat_idx)


def kernel(x, table):
    idx = x[0]
    flat_idx = idx.reshape(1, -1)
    out = _sc_gather(table, flat_idx)
    return out.reshape(idx.shape[0], idx.shape[1], _EMBED_DIM)


# speedup vs baseline: 4.2008x; 1.2907x over previous
import jax
import jax.numpy as jnp
from jax.experimental import pallas as pl
from jax.experimental.pallas import tpu as pltpu
from jax.experimental.pallas import tpu_sc as plsc

_EMBED_DIM = 128
_ROWS = 8


def _sc_gather(table, idx):
    n_batch, n_tok = idx.shape
    vector_mesh = plsc.VectorSubcoreMesh(
        core_axis_name="core", subcore_axis_name="subcore"
    )

    @pl.kernel(
        out_type=jax.ShapeDtypeStruct((n_batch, n_tok, _EMBED_DIM), table.dtype),
        mesh=vector_mesh,
    )
    def gather_kernel(table_hbm, idx_hbm, out_hbm):
        def body(idx_vmem, out_vmem):
            for j in range(_ROWS):
                pltpu.sync_copy(table_hbm.at[idx_vmem.at[j]], out_vmem.at[j])

        pltpu.emit_pipeline(
            body,
            grid=(n_batch // _ROWS,),
            in_specs=[
                pl.BlockSpec((_ROWS, n_tok), index_map=lambda i: (i, 0))
            ],
            out_specs=[
                pl.BlockSpec(
                    (_ROWS, n_tok, _EMBED_DIM), index_map=lambda i: (i, 0, 0)
                )
            ],
            core_axis_name=("core", "subcore"),
            dimension_semantics=(pltpu.PARALLEL,),
        )(idx_hbm, out_hbm)

    return gather_kernel(table, idx)


def kernel(x, table):
    return _sc_gather(table, x[0])


# speedup vs baseline: 5.3789x; 1.2805x over previous
import jax
import jax.numpy as jnp
from jax.experimental import pallas as pl
from jax.experimental.pallas import tpu as pltpu
from jax.experimental.pallas import tpu_sc as plsc

_EMBED_DIM = 128
_GROUP = 8


def _sc_gather(table, idx):
    n_batch, n_tok = idx.shape
    vector_mesh = plsc.VectorSubcoreMesh(
        core_axis_name="core", subcore_axis_name="subcore"
    )
    n_workers = 32
    rows_per_worker = n_batch // n_workers
    n_groups = rows_per_worker // _GROUP

    @pl.kernel(
        out_type=jax.ShapeDtypeStruct((n_batch, n_tok, _EMBED_DIM), table.dtype),
        mesh=vector_mesh,
        scratch_types=[
            pltpu.VMEM((_GROUP, n_tok), jnp.int32),
            pltpu.VMEM((_GROUP, n_tok, _EMBED_DIM), table.dtype),
            pltpu.SemaphoreType.DMA,
            pltpu.SemaphoreType.DMA,
        ],
    )
    def gather_kernel(table_hbm, idx_hbm, out_hbm, idx_v, rows_v, isem, wsem):
        wid = jax.lax.axis_index("subcore") * 2 + jax.lax.axis_index("core")
        base = wid * rows_per_worker

        @pl.loop(0, n_groups)
        def _(t):
            r0 = base + t * _GROUP
            @pl.when(t > 0)
            def _():
                for j in range(_GROUP):
                    pltpu.make_async_copy(
                        rows_v.at[j], out_hbm.at[r0 - _GROUP + j], wsem
                    ).wait()

            pltpu.async_copy(idx_hbm.at[pl.ds(r0, _GROUP)], idx_v, isem
                             ).wait()
            gathers = [
                pltpu.async_copy(table_hbm.at[idx_v.at[j]], rows_v.at[j],
                                 isem)
                for j in range(_GROUP)
            ]
            for g in gathers:
                g.wait()
            for j in range(_GROUP):
                pltpu.async_copy(rows_v.at[j], out_hbm.at[r0 + j], wsem)

        r_last = base + (n_groups - 1) * _GROUP
        for j in range(_GROUP):
            pltpu.make_async_copy(rows_v.at[j], out_hbm.at[r_last + j], wsem
                                  ).wait()

    return gather_kernel(table, idx)


def kernel(x, table):
    return _sc_gather(table, x[0])


# speedup vs baseline: 5.8165x; 1.0814x over previous
import jax
import jax.numpy as jnp
from jax.experimental import pallas as pl
from jax.experimental.pallas import tpu as pltpu
from jax.experimental.pallas import tpu_sc as plsc

_EMBED_DIM = 128
_GROUP = 8


def _sc_gather(table, idx):
    n_batch, n_tok = idx.shape
    vector_mesh = plsc.VectorSubcoreMesh(
        core_axis_name="core", subcore_axis_name="subcore"
    )
    n_workers = 32
    rows_per_worker = n_batch // n_workers
    n_groups = rows_per_worker // _GROUP
    n_pairs = n_groups // 2

    @pl.kernel(
        out_type=jax.ShapeDtypeStruct((n_batch, n_tok, _EMBED_DIM), table.dtype),
        mesh=vector_mesh,
        scratch_types=[
            pltpu.VMEM((_GROUP, n_tok), jnp.int32),
            pltpu.VMEM((_GROUP, n_tok), jnp.int32),
            pltpu.VMEM((_GROUP, n_tok, _EMBED_DIM), table.dtype),
            pltpu.VMEM((_GROUP, n_tok, _EMBED_DIM), table.dtype),
            pltpu.SemaphoreType.DMA,
            pltpu.SemaphoreType.DMA,
            pltpu.SemaphoreType.DMA,
        ],
    )
    def gather_kernel(table_hbm, idx_hbm, out_hbm, idx_v0, idx_v1, rows_v0,
                      rows_v1, isem, gsem, wsem):
        wid = jax.lax.axis_index("subcore") * 2 + jax.lax.axis_index("core")
        base = wid * rows_per_worker
        idx_v = (idx_v0, idx_v1)
        rows_v = (rows_v0, rows_v1)

        def idx_load(g, b):
            return pltpu.async_copy(
                idx_hbm.at[pl.ds(base + g * _GROUP, _GROUP)], idx_v[b], isem
            )

        def fire_gathers(b):
            for j in range(_GROUP):
                pltpu.async_copy(table_hbm.at[idx_v[b].at[j]],
                                 rows_v[b].at[j], gsem)

        def wait_gathers(b):
            for j in range(_GROUP):
                pltpu.make_async_copy(table_hbm.at[idx_v[b].at[j]],
                                      rows_v[b].at[j], gsem).wait()

        def fire_writebacks(g, b):
            r0 = base + g * _GROUP
            for j in range(_GROUP):
                pltpu.async_copy(rows_v[b].at[j], out_hbm.at[r0 + j], wsem)

        def drain_writebacks(g, b):
            r0 = base + g * _GROUP
            for j in range(_GROUP):
                pltpu.make_async_copy(rows_v[b].at[j], out_hbm.at[r0 + j],
                                      wsem).wait()

        idx_load(0, 0).wait()
        fire_gathers(0)
        idx_load(1, 1)

        @pl.loop(0, n_pairs)
        def _(t):
            g0 = 2 * t
            g1 = g0 + 1
            wait_gathers(0)

            @pl.when(t < n_pairs - 1)
            def _():
                idx_load(g0 + 2, 0)

            @pl.when(t > 0)
            def _():
                drain_writebacks(g0 - 1, 1)

            pltpu.make_async_copy(
                idx_hbm.at[pl.ds(base + g1 * _GROUP, _GROUP)], idx_v[1], isem
            ).wait()
            fire_gathers(1)
            fire_writebacks(g0, 0)
            wait_gathers(1)

            @pl.when(t < n_pairs - 1)
            def _():
                idx_load(g1 + 2, 1)

            drain_writebacks(g0, 0)

            @pl.when(t < n_pairs - 1)
            def _():
                pltpu.make_async_copy(
                    idx_hbm.at[pl.ds(base + (g0 + 2) * _GROUP, _GROUP)],
                    idx_v[0], isem
                ).wait()
                fire_gathers(0)

            fire_writebacks(g1, 1)

        drain_writebacks(n_groups - 1, 1)

    return gather_kernel(table, idx)


def kernel(x, table):
    return _sc_gather(table, x[0])
